# baseline (speedup 1.0000x reference)
import functools

import jax
import jax.numpy as jnp
from jax import lax
from jax.experimental import pallas as pl
from jax.experimental.pallas import tpu as pltpu
from jax.experimental.pallas import tpu_sc as plsc

_N_CHUNKS = 4


def kernel(x, src, dim, start, end, step, out):
    m, d = x.shape
    s = src.shape[0]
    info = plsc.get_sparse_core_info()
    nc = info.num_cores
    nw = nc * info.num_subcores
    k = _N_CHUNKS
    src_w = s // nw
    tail_w = (m - s) // nw
    assert s % (nw * k) == 0 and (m - s) % (nw * k) == 0
    mesh = plsc.VectorSubcoreMesh(core_axis_name="c", subcore_axis_name="s")

    @functools.partial(
        pl.kernel,
        mesh=mesh,
        out_type=jax.ShapeDtypeStruct((m, d), x.dtype),
        scratch_types=[pltpu.SemaphoreType.DMA] * (2 * k),
    )
    def run(x_hbm, src_hbm, out_hbm, *sems):
        cid = lax.axis_index("c")
        sid = lax.axis_index("s")
        wid = sid * nc + cid
        src_base = wid * src_w
        tail_base = s + wid * tail_w
        sch = src_w // k
        tch = tail_w // k

        copies = []
        for i in range(k):
            off = src_base + i * sch
            copies.append(
                pltpu.make_async_copy(
                    src_hbm.at[pl.ds(off, sch)], out_hbm.at[pl.ds(off, sch)], sems[i]
                )
            )
        for i in range(k):
            off = tail_base + i * tch
            copies.append(
                pltpu.make_async_copy(
                    x_hbm.at[pl.ds(off, tch)], out_hbm.at[pl.ds(off, tch)], sems[k + i]
                )
            )
        for c in copies:
            c.start()
        for c in copies:
            c.wait()

    return run(x, src)


# speedup vs baseline: 38.2210x; 38.2210x over previous
import functools

import jax
import jax.numpy as jnp
from jax import lax
from jax.experimental import pallas as pl
from jax.experimental.pallas import tpu as pltpu
from jax.experimental.pallas import tpu_sc as plsc

_CHUNK_ROWS = 512


def kernel(x, src, dim, start, end, step, out):
    m, d = x.shape
    s = src.shape[0]
    info = plsc.get_sparse_core_info()
    nc = info.num_cores
    ns = info.num_subcores
    nw = nc * ns
    ch = _CHUNK_ROWS
    src_w = s // nw
    tail_w = (m - s) // nw
    assert s % (nw * ch) == 0 and (m - s) % (nw * ch) == 0
    mesh = plsc.VectorSubcoreMesh(core_axis_name="c", subcore_axis_name="s")

    @functools.partial(
        pl.kernel,
        mesh=mesh,
        out_type=jax.ShapeDtypeStruct((m, d), x.dtype),
        scratch_types=(
            [
                pltpu.VMEM((ch, d), x.dtype),
                pltpu.VMEM_SHARED((ns, ch, d), x.dtype),
            ]
            + [pltpu.SemaphoreType.DMA] * 4
        ),
    )
    def run(x_hbm, src_hbm, out_hbm, tbuf, shared, *sems):
        sems_r = sems[:2]
        sems_w = sems[2:]
        cid = lax.axis_index("c")
        sid = lax.axis_index("s")
        wid = sid * nc + cid
        src_base = wid * src_w
        tail_base = s + wid * tail_w
        bufs = (tbuf, shared.at[sid])

        jobs = [(src_hbm, src_base + i * ch) for i in range(src_w // ch)]
        jobs += [(x_hbm, tail_base + i * ch) for i in range(tail_w // ch)]
        n = len(jobs)

        def rd(i):
            ref, off = jobs[i]
            return pltpu.make_async_copy(
                ref.at[pl.ds(off, ch)], bufs[i % 2], sems_r[i % 2]
            )

        def wr(i):
            off = jobs[i][1]
            return pltpu.make_async_copy(
                bufs[i % 2], out_hbm.at[pl.ds(off, ch)], sems_w[i % 2]
            )

        rd(0).start()
        for i in range(n):
            if i + 1 < n:
                if i >= 1:
                    wr(i - 1).wait()
                rd(i + 1).start()
            rd(i).wait()
            wr(i).start()
        for i in range(max(0, n - 2), n):
            wr(i).wait()

    return run(x, src)


# speedup vs baseline: 38.3462x; 1.0033x over previous
import functools

import jax
import jax.numpy as jnp
from jax import lax
from jax.experimental import pallas as pl
from jax.experimental.pallas import tpu as pltpu
from jax.experimental.pallas import tpu_sc as plsc

_CHUNK_ROWS = 256
_NBUF = 3


def kernel(x, src, dim, start, end, step, out):
    m, d = x.shape
    s = src.shape[0]
    info = plsc.get_sparse_core_info()
    nc = info.num_cores
    ns = info.num_subcores
    nw = nc * ns
    ch = _CHUNK_ROWS
    nb = _NBUF
    src_w = s // nw
    tail_w = (m - s) // nw
    assert s % (nw * ch) == 0 and (m - s) % (nw * ch) == 0
    mesh = plsc.VectorSubcoreMesh(core_axis_name="c", subcore_axis_name="s")

    @functools.partial(
        pl.kernel,
        mesh=mesh,
        out_type=jax.ShapeDtypeStruct((m, d), x.dtype),
        scratch_types=(
            [pltpu.VMEM_SHARED((ns * nb, ch, d), x.dtype)]
            + [pltpu.SemaphoreType.DMA] * (2 * nb)
        ),
    )
    def run(x_hbm, src_hbm, out_hbm, shared, *sems):
        sems_r = sems[:nb]
        sems_w = sems[nb:]
        cid = lax.axis_index("c")
        sid = lax.axis_index("s")
        wid = sid * nc + cid
        src_base = wid * src_w
        tail_base = s + wid * tail_w

        jobs = [(src_hbm, src_base + i * ch) for i in range(src_w // ch)]
        jobs += [(x_hbm, tail_base + i * ch) for i in range(tail_w // ch)]
        n = len(jobs)

        def buf(i):
            return shared.at[sid * nb + (i % nb)]

        def rd(i):
            ref, off = jobs[i]
            return pltpu.make_async_copy(
                ref.at[pl.ds(off, ch)], buf(i), sems_r[i % nb]
            )

        def wr(i):
            off = jobs[i][1]
            return pltpu.make_async_copy(
                buf(i), out_hbm.at[pl.ds(off, ch)], sems_w[i % nb]
            )

        for i in range(min(nb - 1, n)):
            rd(i).start()
        for i in range(n):
            if i + nb - 1 < n:
                if i >= 1:
                    wr(i - 1).wait()
                rd(i + nb - 1).start()
            rd(i).wait()
            wr(i).start()
        for i in range(max(0, n - nb), n):
            wr(i).wait()

    return run(x, src)
